# Initial kernel scaffold; baseline (speedup 1.0000x reference)
import jax
import jax.numpy as jnp
from jax.experimental import pallas as pl


def kernel(x, c_perm, pos_embedding):
    raise NotImplementedError("write your pallas kernel here")


# speedup vs baseline: 4.7349x; 4.7349x over previous
import functools

import jax
import jax.numpy as jnp
from jax import lax
from jax.experimental import pallas as pl
from jax.experimental.pallas import tpu as pltpu
from jax.experimental.pallas import tpu_sc as plsc

BATCH = 16384
N_PERM = 200
N_EMBED = 64
TOTAL = BATCH * N_PERM

CHUNK = 512
GATHER = 128
VREGS = CHUNK // 16


def _build_table_tc(c_perm, pos_embedding):
    def body(pos_ref, cp_ref, out_ref):
        out_ref[...] = pos_ref[...][:, None, :] + cp_ref[...][None, :, :]

    return pl.pallas_call(
        body,
        out_shape=jax.ShapeDtypeStruct((N_PERM, N_PERM, N_EMBED), jnp.float32),
    )(pos_embedding, c_perm)


def _sc_gather(x_flat, table_flat):
    info = plsc.get_sparse_core_info()
    nw = info.num_cores * info.num_subcores
    per_w = TOTAL // nw
    n_chunks = per_w // CHUNK

    mesh = plsc.VectorSubcoreMesh(core_axis_name="c", subcore_axis_name="s")

    @functools.partial(
        pl.kernel,
        out_type=jax.ShapeDtypeStruct((TOTAL, N_EMBED), jnp.float32),
        mesh=mesh,
        scratch_types=[
            pltpu.VMEM((CHUNK,), jnp.int32),
            pltpu.VMEM((CHUNK // GATHER, GATHER), jnp.int32),
            pltpu.VMEM((CHUNK, N_EMBED), jnp.float32),
            pltpu.SemaphoreType.DMA,
        ],
        compiler_params=pltpu.CompilerParams(use_tc_tiling_on_sc=False),
    )
    def k(x_hbm, t_hbm, out_hbm, idx_v, flat_v, rows_v, sem):
        wid = lax.axis_index("s") * info.num_cores + lax.axis_index("c")
        wbase = wid * per_w

        def chunk_body(t, carry):
            base = wbase + t * CHUNK
            pltpu.sync_copy(x_hbm.at[pl.ds(base, CHUNK)], idx_v)
            local0 = t * CHUNK
            iota = lax.iota(jnp.int32, 16)
            for k16 in range(VREGS):
                xv = idx_v[pl.ds(k16 * 16, 16)]
                g = iota + (local0 + k16 * 16)
                p = lax.rem(g, N_PERM)
                flat = p * N_PERM + xv
                flat_v[k16 // (GATHER // 16),
                       pl.ds((k16 % (GATHER // 16)) * 16, 16)] = flat
            copies = [
                pltpu.async_copy(
                    t_hbm.at[flat_v.at[j]],
                    rows_v.at[pl.ds(j * GATHER, GATHER)],
                    sem,
                )
                for j in range(CHUNK // GATHER)
            ]
            for c in copies:
                c.wait()
            pltpu.sync_copy(rows_v, out_hbm.at[pl.ds(base, CHUNK)])
            return carry

        lax.fori_loop(0, n_chunks, chunk_body, 0)

    return k(x_flat, table_flat)


def kernel(x, c_perm, pos_embedding):
    table = _build_table_tc(c_perm, pos_embedding)
    table_flat = table.reshape(N_PERM * N_PERM, N_EMBED)
    x_flat = x.reshape(TOTAL).astype(jnp.int32)
    out = _sc_gather(x_flat, table_flat)
    return out.reshape(BATCH, N_PERM, N_EMBED)


# speedup vs baseline: 5.1955x; 1.0973x over previous
import functools

import jax
import jax.numpy as jnp
from jax import lax
from jax.experimental import pallas as pl
from jax.experimental.pallas import tpu as pltpu
from jax.experimental.pallas import tpu_sc as plsc

BATCH = 16384
N_PERM = 200
N_EMBED = 64
TOTAL = BATCH * N_PERM

CHUNK = 512
GATHER = 128
VREGS = CHUNK // 16


def _build_table_tc(c_perm, pos_embedding):
    def body(pos_ref, cp_ref, out_ref):
        out_ref[...] = pos_ref[...][:, None, :] + cp_ref[...][None, :, :]

    return pl.pallas_call(
        body,
        out_shape=jax.ShapeDtypeStruct((N_PERM, N_PERM, N_EMBED), jnp.float32),
    )(pos_embedding, c_perm)


def _sc_gather(x_flat, table_flat):
    info = plsc.get_sparse_core_info()
    nw = info.num_cores * info.num_subcores
    per_w = TOTAL // nw
    n_chunks = per_w // CHUNK
    n_gath = CHUNK // GATHER

    mesh = plsc.VectorSubcoreMesh(core_axis_name="c", subcore_axis_name="s")

    @functools.partial(
        pl.kernel,
        out_type=jax.ShapeDtypeStruct((TOTAL, N_EMBED), jnp.float32),
        mesh=mesh,
        scratch_types=[
            pltpu.VMEM((2, CHUNK), jnp.int32),
            pltpu.VMEM((2, n_gath, GATHER), jnp.int32),
            pltpu.VMEM((2, CHUNK, N_EMBED), jnp.float32),
            pltpu.SemaphoreType.DMA,
            pltpu.SemaphoreType.DMA,
            pltpu.SemaphoreType.DMA,
            pltpu.SemaphoreType.DMA,
        ],
        compiler_params=pltpu.CompilerParams(use_tc_tiling_on_sc=False),
    )
    def k(x_hbm, t_hbm, out_hbm, idx_v, flat_v, rows_v,
          sem_g0, sem_g1, sem_o0, sem_o1):
        wid = lax.axis_index("s") * info.num_cores + lax.axis_index("c")
        wbase = wid * per_w
        sem_g = (sem_g0, sem_g1)
        sem_o = (sem_o0, sem_o1)
        iota = lax.iota(jnp.int32, 16)

        def stage_indices(t, b):
            base = wbase + t * CHUNK
            pltpu.sync_copy(x_hbm.at[pl.ds(base, CHUNK)], idx_v.at[b])
            local0 = t * CHUNK
            for k16 in range(VREGS):
                xv = idx_v[b, pl.ds(k16 * 16, 16)]
                p = lax.rem(iota + (local0 + k16 * 16), N_PERM)
                flat_v[b, k16 // (GATHER // 16),
                       pl.ds((k16 % (GATHER // 16)) * 16, 16)] = p * N_PERM + xv

        def fire_gathers(b):
            for j in range(n_gath):
                pltpu.async_copy(
                    t_hbm.at[flat_v.at[b].at[j]],
                    rows_v.at[b].at[pl.ds(j * GATHER, GATHER)],
                    sem_g[b],
                )

        def wait_gathers(b):
            for j in range(n_gath):
                pltpu.make_async_copy(
                    t_hbm.at[flat_v.at[b].at[j]],
                    rows_v.at[b].at[pl.ds(j * GATHER, GATHER)],
                    sem_g[b],
                ).wait()

        def fire_outcopy(t, b):
            base = wbase + t * CHUNK
            pltpu.async_copy(rows_v.at[b], out_hbm.at[pl.ds(base, CHUNK)],
                             sem_o[b])

        def wait_outcopy(b):
            pltpu.make_async_copy(rows_v.at[b], out_hbm.at[pl.ds(wbase, CHUNK)],
                                  sem_o[b]).wait()

        stage_indices(0, 0)
        fire_gathers(0)

        def outer(g, carry):
            for b in (0, 1):
                t = 2 * g + b + 1
                bb = (b + 1) % 2

                @pl.when(t < n_chunks)
                def _():
                    stage_indices(t, bb)

                    @pl.when(t >= 2)
                    def _():
                        wait_outcopy(bb)

                    fire_gathers(bb)
                wait_gathers(b)
                fire_outcopy(t - 1, b)
            return carry

        lax.fori_loop(0, (n_chunks + 1) // 2, outer, 0)
        wait_outcopy(0)
        wait_outcopy(1)

    return k(x_flat, table_flat)


def kernel(x, c_perm, pos_embedding):
    table = _build_table_tc(c_perm, pos_embedding)
    table_flat = table.reshape(N_PERM * N_PERM, N_EMBED)
    x_flat = x.reshape(TOTAL).astype(jnp.int32)
    out = _sc_gather(x_flat, table_flat)
    return out.reshape(BATCH, N_PERM, N_EMBED)
